# baseline (device time: 30615 ns/iter reference)
import jax
import jax.numpy as jnp
from jax import lax
from jax.experimental import pallas as pl
from jax.experimental.pallas import tpu as pltpu


def kernel(x, router, W1, W2):
    T_loc, D = x.shape
    E_loc = W1.shape[0]
    F = W1.shape[2]
    T = 2 * T_loc

    def body(x_ref, r_ref, w1_ref, w2_ref, out_ref,
             xfull, rpeer, pfull, pin, sems):
        my_x = lax.axis_index("x")
        my_y = lax.axis_index("y")
        my_z = lax.axis_index("z")
        peer = (1 - my_x, my_y, my_z)

        barrier = pltpu.get_barrier_semaphore()
        pl.semaphore_signal(barrier, inc=1, device_id=peer,
                            device_id_type=pl.DeviceIdType.MESH)
        pl.semaphore_wait(barrier, 1)

        rdma_x = pltpu.make_async_remote_copy(
            src_ref=x_ref,
            dst_ref=xfull.at[pl.ds(my_x * T_loc, T_loc)],
            send_sem=sems.at[0], recv_sem=sems.at[1],
            device_id=peer, device_id_type=pl.DeviceIdType.MESH,
        )
        rdma_x.start()
        rdma_r = pltpu.make_async_remote_copy(
            src_ref=r_ref, dst_ref=rpeer,
            send_sem=sems.at[2], recv_sem=sems.at[3],
            device_id=peer, device_id_type=pl.DeviceIdType.MESH,
        )
        rdma_r.start()

        xfull[pl.ds(my_x * T_loc, T_loc), :] = x_ref[:, :]

        rdma_x.wait()
        rdma_r.wait()

        xf = xfull[:, :]
        gl = jnp.dot(xf, r_ref[:, :], preferred_element_type=jnp.float32)
        gr = jnp.dot(xf, rpeer[:, :], preferred_element_type=jnp.float32)

        g0, g1 = gl[:, 0:1], gl[:, 1:2]
        g2, g3 = gr[:, 0:1], gr[:, 1:2]
        a = jnp.maximum(g0, g1)
        b = jnp.minimum(g0, g1)
        c = jnp.maximum(g2, g3)
        d = jnp.minimum(g2, g3)
        m1 = jnp.maximum(a, c)
        m2 = jnp.where(a >= c, jnp.maximum(b, c), jnp.maximum(d, a))

        def wgt(g):
            return jnp.where(g >= m2, jnp.exp(g - m1), 0.0)

        t0, t1, t2, t3 = wgt(g0), wgt(g1), wgt(g2), wgt(g3)
        z = t0 + t1 + t2 + t3
        w_local = [t0 / z, t1 / z]

        xb = xf.astype(jnp.bfloat16)
        acc = jnp.zeros((T, D), jnp.float32)
        for e in range(E_loc):
            h = jnp.dot(xb, w1_ref[e].astype(jnp.bfloat16),
                        preferred_element_type=jnp.float32)
            h = jnp.maximum(h, 0.0).astype(jnp.bfloat16)
            o = jnp.dot(h, w2_ref[e].astype(jnp.bfloat16),
                        preferred_element_type=jnp.float32)
            acc = acc + o * w_local[e]
        pfull[:, :] = acc

        rdma_p = pltpu.make_async_remote_copy(
            src_ref=pfull.at[pl.ds((1 - my_x) * T_loc, T_loc)],
            dst_ref=pin,
            send_sem=sems.at[4], recv_sem=sems.at[5],
            device_id=peer, device_id_type=pl.DeviceIdType.MESH,
        )
        rdma_p.start()
        rdma_p.wait()

        out_ref[:, :] = pfull[pl.ds(my_x * T_loc, T_loc), :] + pin[:, :]

    return pl.pallas_call(
        body,
        out_shape=jax.ShapeDtypeStruct((T_loc, D), jnp.float32),
        in_specs=[
            pl.BlockSpec(memory_space=pltpu.VMEM),
            pl.BlockSpec(memory_space=pltpu.VMEM),
            pl.BlockSpec(memory_space=pltpu.VMEM),
            pl.BlockSpec(memory_space=pltpu.VMEM),
        ],
        out_specs=pl.BlockSpec(memory_space=pltpu.VMEM),
        scratch_shapes=[
            pltpu.VMEM((T, D), jnp.float32),
            pltpu.VMEM((D, E_loc), jnp.float32),
            pltpu.VMEM((T, D), jnp.float32),
            pltpu.VMEM((T_loc, D), jnp.float32),
            pltpu.SemaphoreType.DMA((6,)),
        ],
        compiler_params=pltpu.CompilerParams(collective_id=0),
    )(x, router, W1, W2)


# device time: 26560 ns/iter; 1.1527x vs baseline; 1.1527x over previous
import jax
import jax.numpy as jnp
from jax import lax
from jax.experimental import pallas as pl
from jax.experimental.pallas import tpu as pltpu


def kernel(x, router, W1, W2):
    T_loc, D = x.shape
    E_loc = W1.shape[0]

    def body(x_ref, r_ref, w1_ref, w2_ref, out_ref,
             xsend, xin, rpeer, wsend, win, psend, pin, sems):
        my_x = lax.axis_index("x")
        my_y = lax.axis_index("y")
        my_z = lax.axis_index("z")
        peer = (1 - my_x, my_y, my_z)

        barrier = pltpu.get_barrier_semaphore()
        pl.semaphore_signal(barrier, inc=1, device_id=peer,
                            device_id_type=pl.DeviceIdType.MESH)
        pl.semaphore_wait(barrier, 1)

        xsend[:, :] = x_ref[:, :].astype(jnp.bfloat16)
        rdma_x = pltpu.make_async_remote_copy(
            src_ref=xsend, dst_ref=xin,
            send_sem=sems.at[0], recv_sem=sems.at[1],
            device_id=peer, device_id_type=pl.DeviceIdType.MESH,
        )
        rdma_x.start()
        rdma_r = pltpu.make_async_remote_copy(
            src_ref=r_ref, dst_ref=rpeer,
            send_sem=sems.at[2], recv_sem=sems.at[3],
            device_id=peer, device_id_type=pl.DeviceIdType.MESH,
        )
        rdma_r.start()
        rdma_r.wait()

        xl = x_ref[:, :]
        gl = jnp.dot(xl, r_ref[:, :], preferred_element_type=jnp.float32)
        gr = jnp.dot(xl, rpeer[:, :], preferred_element_type=jnp.float32)

        g0, g1 = gl[:, 0:1], gl[:, 1:2]
        g2, g3 = gr[:, 0:1], gr[:, 1:2]
        a = jnp.maximum(g0, g1)
        b = jnp.minimum(g0, g1)
        c = jnp.maximum(g2, g3)
        d = jnp.minimum(g2, g3)
        m1 = jnp.maximum(a, c)
        m2 = jnp.where(a >= c, jnp.maximum(b, c), jnp.maximum(d, a))

        def wgt(g):
            return jnp.where(g >= m2, jnp.exp(g - m1), 0.0)

        t0, t1, t2, t3 = wgt(g0), wgt(g1), wgt(g2), wgt(g3)
        z = t0 + t1 + t2 + t3
        w_mine = [t0 / z, t1 / z]

        wsend[:, :] = jnp.concatenate([t2 / z, t3 / z], axis=1)
        rdma_w = pltpu.make_async_remote_copy(
            src_ref=wsend, dst_ref=win,
            send_sem=sems.at[4], recv_sem=sems.at[5],
            device_id=peer, device_id_type=pl.DeviceIdType.MESH,
        )
        rdma_w.start()

        xbl = xl.astype(jnp.bfloat16)
        acc_my = jnp.zeros((T_loc, D), jnp.float32)
        for e in range(E_loc):
            h = jnp.dot(xbl, w1_ref[e].astype(jnp.bfloat16),
                        preferred_element_type=jnp.float32)
            h = jnp.maximum(h, 0.0).astype(jnp.bfloat16)
            o = jnp.dot(h, w2_ref[e].astype(jnp.bfloat16),
                        preferred_element_type=jnp.float32)
            acc_my = acc_my + o * w_mine[e]

        rdma_x.wait()
        rdma_w.wait()
        xp = xin[:, :]
        acc_peer = jnp.zeros((T_loc, D), jnp.float32)
        for e in range(E_loc):
            h = jnp.dot(xp, w1_ref[e].astype(jnp.bfloat16),
                        preferred_element_type=jnp.float32)
            h = jnp.maximum(h, 0.0).astype(jnp.bfloat16)
            o = jnp.dot(h, w2_ref[e].astype(jnp.bfloat16),
                        preferred_element_type=jnp.float32)
            acc_peer = acc_peer + o * win[:, e:e + 1]

        psend[:, :] = acc_peer.astype(jnp.bfloat16)
        rdma_p = pltpu.make_async_remote_copy(
            src_ref=psend, dst_ref=pin,
            send_sem=sems.at[6], recv_sem=sems.at[7],
            device_id=peer, device_id_type=pl.DeviceIdType.MESH,
        )
        rdma_p.start()
        rdma_p.wait()

        out_ref[:, :] = acc_my + pin[:, :].astype(jnp.float32)

    return pl.pallas_call(
        body,
        out_shape=jax.ShapeDtypeStruct((T_loc, D), jnp.float32),
        in_specs=[
            pl.BlockSpec(memory_space=pltpu.VMEM),
            pl.BlockSpec(memory_space=pltpu.VMEM),
            pl.BlockSpec(memory_space=pltpu.VMEM),
            pl.BlockSpec(memory_space=pltpu.VMEM),
        ],
        out_specs=pl.BlockSpec(memory_space=pltpu.VMEM),
        scratch_shapes=[
            pltpu.VMEM((T_loc, D), jnp.bfloat16),
            pltpu.VMEM((T_loc, D), jnp.bfloat16),
            pltpu.VMEM((D, E_loc), jnp.float32),
            pltpu.VMEM((T_loc, E_loc), jnp.float32),
            pltpu.VMEM((T_loc, E_loc), jnp.float32),
            pltpu.VMEM((T_loc, D), jnp.bfloat16),
            pltpu.VMEM((T_loc, D), jnp.bfloat16),
            pltpu.SemaphoreType.DMA((8,)),
        ],
        compiler_params=pltpu.CompilerParams(collective_id=0),
    )(x, router, W1, W2)
